# baseline (speedup 1.0000x reference)
import functools

import jax
import jax.numpy as jnp
from jax import lax
from jax.experimental import pallas as pl
from jax.experimental.pallas import tpu as pltpu
from jax.experimental.pallas import tpu_sc as plsc

_B = 16384
_DIM = 32
_NC = 2
_NS = 16
_NW = _NC * _NS
_BPW = _B // _NW
_NB = 8
_L = 16
_RPAD = 544
_OUTE = (_B + _L) * _DIM


def _gather_body(idx_hbm, pos_hbm, tab_hbm, out_hbm,
                 idxv, posv, rcol, rstart, rend, slabs, stag, semr, semo):
    wid = lax.axis_index("s") * _NC + lax.axis_index("c")
    base = wid * _BPW
    lane = lax.iota(jnp.int32, _L)

    pltpu.sync_copy(idx_hbm.at[pl.ds(base, _BPW)], idxv)
    pltpu.sync_copy(pos_hbm.at[pl.ds(base, _BPW)], posv)

    def scan(v, runbase):
        ch = v * _L + lane
        iv = idxv[pl.ds(pl.multiple_of(v * _L, _L), _L)]
        col = lax.shift_right_logical(iv, 7)
        prev = lax.shift_right_logical(
            plsc.load_gather(idxv, [jnp.maximum(ch - 1, 0)]), 7)
        isstart = (ch == 0) | (col != prev)
        rid = plsc.cumsum(isstart.astype(jnp.int32)) + runbase
        plsc.store_scatter(rcol, [rid - 1], col, mask=isstart)
        plsc.store_scatter(rstart, [rid - 1], ch, mask=isstart)
        endmask = isstart & (rid >= 2)
        plsc.store_scatter(rend, [jnp.maximum(rid - 2, 0)], ch, mask=endmask)
        return rid[_L - 1]

    nruns = lax.fori_loop(0, _BPW // _L, scan, jnp.int32(0))
    plsc.store_scatter(rend, [jnp.full((_L,), nruns - 1, jnp.int32)],
                       jnp.full((_L,), _BPW, jnp.int32), mask=lane == 0)

    def runmeta(g8):
        i8 = g8 * _NB + lane
        return (plsc.load_gather(rcol, [i8]),
                plsc.load_gather(rstart, [i8]),
                plsc.load_gather(rend, [i8]))

    def fire(colv, b):
        cb = pl.multiple_of(colv * 128, 128)
        pltpu.async_copy(tab_hbm.at[:, pl.ds(cb, 128)], slabs[b], semr.at[b])

    def drain_slab(b):
        pltpu.make_async_copy(tab_hbm.at[:, pl.ds(0, 128)], slabs[b],
                              semr.at[b]).wait()

    cols0, _, _ = runmeta(0)
    for b in range(_NB):
        @pl.when(b < nruns)
        def _():
            fire(cols0[b], b)

    def process(b, st, en, chcnt):
        def chunk(k, chc):
            ch = st + k * _L + lane
            m = ch < en
            chc_ = jnp.where(m, ch, 0)
            lans = plsc.load_gather(idxv, [chc_]) & 127
            poss = jnp.where(m, plsc.load_gather(posv, [chc_]), jnp.int32(_B))

            @pl.when(chc >= 4)
            def _():
                for _e in range(_L):
                    pltpu.make_async_copy(stag.at[pl.ds(0, _DIM)],
                                          out_hbm.at[pl.ds(0, _DIM)],
                                          semo).wait()

            sbase = (chc & 3) * (_L * _DIM)
            for d in range(_DIM):
                vals = plsc.load_gather(
                    slabs[b], [jnp.full((_L,), d, jnp.int32), lans])
                plsc.store_scatter(stag, [sbase + lane * _DIM + d], vals)
            for e in range(_L):
                p = poss[e]
                pltpu.async_copy(
                    stag.at[pl.ds(pl.multiple_of(sbase + e * _DIM, _DIM), _DIM)],
                    out_hbm.at[pl.ds(pl.multiple_of(p * _DIM, _DIM), _DIM)],
                    semo)
            return chc + 1

        nch = lax.shift_right_logical(en - st + (_L - 1), 4)
        return lax.fori_loop(0, nch, chunk, chcnt)

    def group(g8, chcnt):
        cols, sts, ens = runmeta(g8)
        colsn, _, _ = runmeta(g8 + 1)
        for b in range(_NB):
            r = g8 * _NB + b
            valid = r < nruns

            @pl.when(valid)
            def _():
                drain_slab(b)

            st = jnp.where(valid, sts[b], 0)
            en = jnp.where(valid, ens[b], 0)
            chcnt = process(b, st, en, chcnt)

            @pl.when(r + _NB < nruns)
            def _():
                fire(colsn[b], b)
        return chcnt

    ngroups = lax.shift_right_logical(nruns + (_NB - 1), 3)
    chcnt = lax.fori_loop(0, ngroups, group, jnp.int32(0))

    def fin(_i, c):
        pltpu.make_async_copy(stag.at[pl.ds(0, _DIM)],
                              out_hbm.at[pl.ds(0, _DIM)], semo).wait()
        return c
    lax.fori_loop(0, jnp.minimum(chcnt, 4) * _L, fin, 0)


@functools.partial(
    pl.kernel,
    out_type=jax.ShapeDtypeStruct((_OUTE,), jnp.float32),
    mesh=plsc.VectorSubcoreMesh(core_axis_name="c", subcore_axis_name="s"),
    scratch_types=[
        pltpu.VMEM((_BPW,), jnp.int32),
        pltpu.VMEM((_BPW,), jnp.int32),
        pltpu.VMEM((_RPAD,), jnp.int32),
        pltpu.VMEM((_RPAD,), jnp.int32),
        pltpu.VMEM((_RPAD,), jnp.int32),
        [pltpu.VMEM((_DIM, 128), jnp.float32)] * _NB,
        pltpu.VMEM((4 * _L * _DIM,), jnp.float32),
        pltpu.SemaphoreType.DMA((_NB,)),
        pltpu.SemaphoreType.DMA,
    ],
    compiler_params=pltpu.CompilerParams(needs_layout_passes=False),
)
def _gather_rows(idx_hbm, pos_hbm, tab_hbm, out_hbm, *scratch):
    _gather_body(idx_hbm, pos_hbm, tab_hbm, out_hbm, *scratch)


def _dots_body(s_hbm, t_hbm, out_hbm, srow, trow, outv):
    wid = lax.axis_index("s") * _NC + lax.axis_index("c")
    base = wid * _BPW
    pltpu.sync_copy(
        s_hbm.at[pl.ds(pl.multiple_of(base * _DIM, _L), _BPW * _DIM)], srow)
    pltpu.sync_copy(
        t_hbm.at[pl.ds(pl.multiple_of(base * _DIM, _L), _BPW * _DIM)], trow)
    lane = lax.iota(jnp.int32, _L)

    def group(g, c):
        e0 = (g * _L + lane) * _DIM
        acc = jnp.zeros((_L,), jnp.float32)
        for d in range(_DIM):
            sv = plsc.load_gather(srow, [e0 + d])
            tv = plsc.load_gather(trow, [e0 + d])
            acc = acc + sv * tv
        outv[pl.ds(pl.multiple_of(g * _L, _L), _L)] = acc
        return c

    lax.fori_loop(0, _BPW // _L, group, 0)
    pltpu.sync_copy(outv, out_hbm.at[pl.ds(pl.multiple_of(base, _L), _BPW)])


@functools.partial(
    pl.kernel,
    out_type=jax.ShapeDtypeStruct((_B,), jnp.float32),
    mesh=plsc.VectorSubcoreMesh(core_axis_name="c", subcore_axis_name="s"),
    scratch_types=[
        pltpu.VMEM((_BPW * _DIM,), jnp.float32),
        pltpu.VMEM((_BPW * _DIM,), jnp.float32),
        pltpu.VMEM((_BPW,), jnp.float32),
    ],
    compiler_params=pltpu.CompilerParams(needs_layout_passes=False),
)
def _dots(s_hbm, t_hbm, out_hbm, srow, trow, outv):
    _dots_body(s_hbm, t_hbm, out_hbm, srow, trow, outv)


def _loss_body(ip_ref, lab_ref, o_ref):
    x = lab_ref[...] * ip_ref[...]
    o_ref[0, 0] = -jnp.sum(jax.nn.log_sigmoid(x)) * (1.0 / _B)


_loss = pl.pallas_call(
    _loss_body,
    out_shape=jax.ShapeDtypeStruct((1, 1), jnp.float32),
    out_specs=pl.BlockSpec(memory_space=pltpu.MemorySpace.SMEM),
)


def kernel(source_node, target_node, label, nodes_embed, context_nodes_embed):
    iota = jnp.arange(_B, dtype=jnp.int32)
    ss, sp = lax.sort_key_val(source_node, iota)
    ts, tp = lax.sort_key_val(target_node, iota)
    srows = _gather_rows(ss, sp, nodes_embed.T)
    trows = _gather_rows(ts, tp, context_nodes_embed.T)
    ip = _dots(srows, trows)
    loss = _loss(ip.reshape(128, 128), label.reshape(128, 128))
    return loss.reshape(())


# speedup vs baseline: 2.4827x; 2.4827x over previous
import functools

import jax
import jax.numpy as jnp
from jax import lax
from jax.experimental import pallas as pl
from jax.experimental.pallas import tpu as pltpu
from jax.experimental.pallas import tpu_sc as plsc

_B = 16384
_DIM = 32
_NC = 2
_NS = 16
_NW = _NC * _NS
_BPW = _B // _NW
_NB = 8
_L = 16
_RPAD = 544
_OUTE = (_B + _L) * _DIM


def _gather_body(idx_hbm, pos_hbm, tab_hbm, out_hbm,
                 idxv, posv, rcol, rstart, rend, slabs, stag, semr, semo):
    wid = lax.axis_index("s") * _NC + lax.axis_index("c")
    base = wid * _BPW
    lane = lax.iota(jnp.int32, _L)

    pltpu.sync_copy(idx_hbm.at[pl.ds(base, _BPW)], idxv)
    pltpu.sync_copy(pos_hbm.at[pl.ds(base, _BPW)], posv)

    def scan(v, runbase):
        ch = v * _L + lane
        iv = idxv[pl.ds(pl.multiple_of(v * _L, _L), _L)]
        col = lax.shift_right_logical(iv, 7)
        prev = lax.shift_right_logical(
            plsc.load_gather(idxv, [jnp.maximum(ch - 1, 0)]), 7)
        isstart = (ch == 0) | (col != prev)
        rid = plsc.cumsum(isstart.astype(jnp.int32)) + runbase
        plsc.store_scatter(rcol, [rid - 1], col, mask=isstart)
        plsc.store_scatter(rstart, [rid - 1], ch, mask=isstart)
        endmask = isstart & (rid >= 2)
        plsc.store_scatter(rend, [jnp.maximum(rid - 2, 0)], ch, mask=endmask)
        return rid[_L - 1]

    nruns = lax.fori_loop(0, _BPW // _L, scan, jnp.int32(0))
    plsc.store_scatter(rend, [jnp.full((_L,), nruns - 1, jnp.int32)],
                       jnp.full((_L,), _BPW, jnp.int32), mask=lane == 0)

    def runmeta(g8):
        i8 = g8 * _NB + lane
        return (plsc.load_gather(rcol, [i8]),
                plsc.load_gather(rstart, [i8]),
                plsc.load_gather(rend, [i8]))

    def fire(colv, b):
        cb = pl.multiple_of(colv * 128, 128)
        pltpu.async_copy(tab_hbm.at[:, pl.ds(cb, 128)], slabs[b], semr.at[b])

    def drain_slab(b):
        pltpu.make_async_copy(tab_hbm.at[:, pl.ds(0, 128)], slabs[b],
                              semr.at[b]).wait()

    cols0, _, _ = runmeta(0)
    for b in range(_NB):
        @pl.when(b < nruns)
        def _():
            fire(cols0[b], b)

    def process(b, st, en, chcnt):
        def chunk(k, chc):
            ch = st + k * _L + lane
            m = ch < en
            chc_ = jnp.where(m, ch, 0)
            lans = plsc.load_gather(idxv, [chc_]) & 127
            poss = jnp.where(m, plsc.load_gather(posv, [chc_]), _B + lane)

            @pl.when(chc >= 4)
            def _():
                for _e in range(_L):
                    pltpu.make_async_copy(stag.at[pl.ds(0, _DIM)],
                                          out_hbm.at[pl.ds(0, _DIM)],
                                          semo).wait()

            sbase = (chc & 3) * (_L * _DIM)
            for d in range(_DIM):
                vals = plsc.load_gather(
                    slabs[b], [jnp.full((_L,), d, jnp.int32), lans])
                plsc.store_scatter(stag, [sbase + lane * _DIM + d], vals)
            for e in range(_L):
                p = poss[e]
                pltpu.async_copy(
                    stag.at[pl.ds(pl.multiple_of(sbase + e * _DIM, _DIM), _DIM)],
                    out_hbm.at[pl.ds(pl.multiple_of(p * _DIM, _DIM), _DIM)],
                    semo)
            return chc + 1

        nch = lax.shift_right_logical(en - st + (_L - 1), 4)
        return lax.fori_loop(0, nch, chunk, chcnt)

    def group(g8, chcnt):
        cols, sts, ens = runmeta(g8)
        colsn, _, _ = runmeta(g8 + 1)
        for b in range(_NB):
            r = g8 * _NB + b
            valid = r < nruns

            @pl.when(valid)
            def _():
                drain_slab(b)

            st = jnp.where(valid, sts[b], 0)
            en = jnp.where(valid, ens[b], 0)
            chcnt = process(b, st, en, chcnt)

            @pl.when(r + _NB < nruns)
            def _():
                fire(colsn[b], b)
        return chcnt

    ngroups = lax.shift_right_logical(nruns + (_NB - 1), 3)
    chcnt = lax.fori_loop(0, ngroups, group, jnp.int32(0))

    def fin(_i, c):
        pltpu.make_async_copy(stag.at[pl.ds(0, _DIM)],
                              out_hbm.at[pl.ds(0, _DIM)], semo).wait()
        return c
    lax.fori_loop(0, jnp.minimum(chcnt, 4) * _L, fin, 0)


@functools.partial(
    pl.kernel,
    out_type=jax.ShapeDtypeStruct((_OUTE,), jnp.float32),
    mesh=plsc.VectorSubcoreMesh(core_axis_name="c", subcore_axis_name="s"),
    scratch_types=[
        pltpu.VMEM((_BPW,), jnp.int32),
        pltpu.VMEM((_BPW,), jnp.int32),
        pltpu.VMEM((_RPAD,), jnp.int32),
        pltpu.VMEM((_RPAD,), jnp.int32),
        pltpu.VMEM((_RPAD,), jnp.int32),
        [pltpu.VMEM((_DIM, 128), jnp.float32)] * _NB,
        pltpu.VMEM((4 * _L * _DIM,), jnp.float32),
        pltpu.SemaphoreType.DMA((_NB,)),
        pltpu.SemaphoreType.DMA,
    ],
    compiler_params=pltpu.CompilerParams(needs_layout_passes=False),
)
def _gather_rows(idx_hbm, pos_hbm, tab_hbm, out_hbm, *scratch):
    _gather_body(idx_hbm, pos_hbm, tab_hbm, out_hbm, *scratch)


def _dots_body(s_hbm, t_hbm, out_hbm, srow, trow, outv):
    wid = lax.axis_index("s") * _NC + lax.axis_index("c")
    base = wid * _BPW
    pltpu.sync_copy(
        s_hbm.at[pl.ds(pl.multiple_of(base * _DIM, _L), _BPW * _DIM)], srow)
    pltpu.sync_copy(
        t_hbm.at[pl.ds(pl.multiple_of(base * _DIM, _L), _BPW * _DIM)], trow)
    lane = lax.iota(jnp.int32, _L)

    def group(g, c):
        e0 = (g * _L + lane) * _DIM
        acc = jnp.zeros((_L,), jnp.float32)
        for d in range(_DIM):
            sv = plsc.load_gather(srow, [e0 + d])
            tv = plsc.load_gather(trow, [e0 + d])
            acc = acc + sv * tv
        outv[pl.ds(pl.multiple_of(g * _L, _L), _L)] = acc
        return c

    lax.fori_loop(0, _BPW // _L, group, 0)
    pltpu.sync_copy(outv, out_hbm.at[pl.ds(pl.multiple_of(base, _L), _BPW)])


@functools.partial(
    pl.kernel,
    out_type=jax.ShapeDtypeStruct((_B,), jnp.float32),
    mesh=plsc.VectorSubcoreMesh(core_axis_name="c", subcore_axis_name="s"),
    scratch_types=[
        pltpu.VMEM((_BPW * _DIM,), jnp.float32),
        pltpu.VMEM((_BPW * _DIM,), jnp.float32),
        pltpu.VMEM((_BPW,), jnp.float32),
    ],
    compiler_params=pltpu.CompilerParams(needs_layout_passes=False),
)
def _dots(s_hbm, t_hbm, out_hbm, srow, trow, outv):
    _dots_body(s_hbm, t_hbm, out_hbm, srow, trow, outv)


def _loss_body(ip_ref, lab_ref, o_ref):
    x = lab_ref[...] * ip_ref[...]
    o_ref[0, 0] = -jnp.sum(jax.nn.log_sigmoid(x)) * (1.0 / _B)


_loss = pl.pallas_call(
    _loss_body,
    out_shape=jax.ShapeDtypeStruct((1, 1), jnp.float32),
    out_specs=pl.BlockSpec(memory_space=pltpu.MemorySpace.SMEM),
)


def kernel(source_node, target_node, label, nodes_embed, context_nodes_embed):
    iota = jnp.arange(_B, dtype=jnp.int32)
    ss, sp = lax.sort_key_val(source_node, iota)
    ts, tp = lax.sort_key_val(target_node, iota)
    srows = _gather_rows(ss, sp, nodes_embed.T)
    trows = _gather_rows(ts, tp, context_nodes_embed.T)
    ip = _dots(srows, trows)
    loss = _loss(ip.reshape(128, 128), label.reshape(128, 128))
    return loss.reshape(())


# speedup vs baseline: 8.1762x; 3.2933x over previous
import functools

import jax
import jax.numpy as jnp
from jax import lax
from jax.experimental import pallas as pl
from jax.experimental.pallas import tpu as pltpu
from jax.experimental.pallas import tpu_sc as plsc

_B = 16384
_DIM = 32
_NC = 2
_NS = 16
_NW = _NC * _NS
_BPW = _B // _NW
_NB = 8
_L = 16
_RPAD = 544


def _gather_body(idx_hbm, tab_hbm, out_hbm,
                 idxv, rcol, rstart, rend, slabs, stag, semr, semo):
    wid = lax.axis_index("s") * _NC + lax.axis_index("c")
    base = wid * _BPW
    lane = lax.iota(jnp.int32, _L)

    pltpu.sync_copy(idx_hbm.at[pl.ds(base, _BPW)], idxv)

    def scan(v, runbase):
        ch = v * _L + lane
        iv = idxv[pl.ds(pl.multiple_of(v * _L, _L), _L)]
        col = lax.shift_right_logical(iv, 7)
        prev = lax.shift_right_logical(
            plsc.load_gather(idxv, [jnp.maximum(ch - 1, 0)]), 7)
        isstart = (ch == 0) | (col != prev)
        rid = plsc.cumsum(isstart.astype(jnp.int32)) + runbase
        plsc.store_scatter(rcol, [rid - 1], col, mask=isstart)
        plsc.store_scatter(rstart, [rid - 1], ch, mask=isstart)
        endmask = isstart & (rid >= 2)
        plsc.store_scatter(rend, [jnp.maximum(rid - 2, 0)], ch, mask=endmask)
        return rid[_L - 1]

    nruns = lax.fori_loop(0, _BPW // _L, scan, jnp.int32(0))
    plsc.store_scatter(rend, [jnp.full((_L,), nruns - 1, jnp.int32)],
                       jnp.full((_L,), _BPW, jnp.int32), mask=lane == 0)

    def runmeta(g8):
        i8 = g8 * _NB + lane
        return (plsc.load_gather(rcol, [i8]),
                plsc.load_gather(rstart, [i8]),
                plsc.load_gather(rend, [i8]))

    def fire(colv, b):
        cb = pl.multiple_of(colv * 128, 128)
        pltpu.async_copy(tab_hbm.at[:, pl.ds(cb, 128)], slabs[b], semr.at[b])

    def drain_slab(b):
        pltpu.make_async_copy(tab_hbm.at[:, pl.ds(0, 128)], slabs[b],
                              semr.at[b]).wait()

    cols0, _, _ = runmeta(0)
    for b in range(_NB):
        @pl.when(b < nruns)
        def _():
            fire(cols0[b], b)

    def process(b, st, en, nf):
        def chunk(k, n):
            a = st + k * _L
            ch = a + lane
            m = ch < en
            chc_ = jnp.where(m, ch, 0)
            lans = plsc.load_gather(idxv, [chc_]) & 127
            tgt = (lax.shift_right_logical(chc_, 4) & 3) * (_L * _DIM) \
                + (chc_ & (_L - 1)) * _DIM
            for d in range(_DIM):
                vals = plsc.load_gather(
                    slabs[b], [jnp.full((_L,), d, jnp.int32), lans])
                plsc.store_scatter(stag, [tgt + d], vals, mask=m)

            edge = a | (_L - 1)

            def fire_block(n2):
                @pl.when(n2 >= 3)
                def _():
                    pltpu.make_async_copy(
                        stag.at[pl.ds(0, _L * _DIM)],
                        out_hbm.at[pl.ds(0, _L * _DIM)], semo).wait()
                blk = lax.shift_right_logical(edge, 4)
                sb = pl.multiple_of((blk & 3) * (_L * _DIM), _L * _DIM)
                dst = pl.multiple_of(base * _DIM + blk * (_L * _DIM),
                                     _L * _DIM)
                pltpu.async_copy(stag.at[pl.ds(sb, _L * _DIM)],
                                 out_hbm.at[pl.ds(dst, _L * _DIM)], semo)
                return n2 + 1

            return lax.cond(edge < en, fire_block, lambda n2: n2, n)

        nch = lax.shift_right_logical(en - st + (_L - 1), 4)
        return lax.fori_loop(0, nch, chunk, nf)

    def group(g8, nf):
        cols, sts, ens = runmeta(g8)
        colsn, _, _ = runmeta(g8 + 1)
        for b in range(_NB):
            r = g8 * _NB + b
            valid = r < nruns

            @pl.when(valid)
            def _():
                drain_slab(b)

            st = jnp.where(valid, sts[b], 0)
            en = jnp.where(valid, ens[b], 0)
            nf = process(b, st, en, nf)

            @pl.when(r + _NB < nruns)
            def _():
                fire(colsn[b], b)
        return nf

    ngroups = lax.shift_right_logical(nruns + (_NB - 1), 3)
    nf = lax.fori_loop(0, ngroups, group, jnp.int32(0))

    def fin(_i, c):
        pltpu.make_async_copy(stag.at[pl.ds(0, _L * _DIM)],
                              out_hbm.at[pl.ds(0, _L * _DIM)], semo).wait()
        return c
    lax.fori_loop(0, jnp.minimum(nf, 3), fin, 0)


@functools.partial(
    pl.kernel,
    out_type=jax.ShapeDtypeStruct((_B * _DIM,), jnp.float32),
    mesh=plsc.VectorSubcoreMesh(core_axis_name="c", subcore_axis_name="s"),
    scratch_types=[
        pltpu.VMEM((_BPW,), jnp.int32),
        pltpu.VMEM((_RPAD,), jnp.int32),
        pltpu.VMEM((_RPAD,), jnp.int32),
        pltpu.VMEM((_RPAD,), jnp.int32),
        [pltpu.VMEM((_DIM, 128), jnp.float32)] * _NB,
        pltpu.VMEM((4 * _L * _DIM,), jnp.float32),
        pltpu.SemaphoreType.DMA((_NB,)),
        pltpu.SemaphoreType.DMA,
    ],
    compiler_params=pltpu.CompilerParams(needs_layout_passes=False),
)
def _gather_rows(idx_hbm, tab_hbm, out_hbm, *scratch):
    _gather_body(idx_hbm, tab_hbm, out_hbm, *scratch)


def _dots_body(s_hbm, t_hbm, sinv_hbm, tinv_hbm, out_hbm,
               sinvv, tinvv, sbufs, tbufs, outv, sems, semt):
    wid = lax.axis_index("s") * _NC + lax.axis_index("c")
    base = wid * _BPW
    lane = lax.iota(jnp.int32, _L)

    pltpu.sync_copy(sinv_hbm.at[pl.ds(base, _BPW)], sinvv)
    pltpu.sync_copy(tinv_hbm.at[pl.ds(base, _BPW)], tinvv)

    def fire(rs, rt, b):
        so = pl.multiple_of(rs * _DIM, _DIM)
        to = pl.multiple_of(rt * _DIM, _DIM)
        pltpu.async_copy(s_hbm.at[pl.ds(so, _DIM)], sbufs[b], sems.at[b])
        pltpu.async_copy(t_hbm.at[pl.ds(to, _DIM)], tbufs[b], semt.at[b])

    def drain(b):
        pltpu.make_async_copy(s_hbm.at[pl.ds(0, _DIM)], sbufs[b], sems.at[b]).wait()
        pltpu.make_async_copy(t_hbm.at[pl.ds(0, _DIM)], tbufs[b], semt.at[b]).wait()

    def idx_vecs(g):
        off = pl.multiple_of(g * _L, _L)
        return sinvv[pl.ds(off, _L)], tinvv[pl.ds(off, _L)]

    siv0, tiv0 = idx_vecs(0)
    for b in range(_NB):
        fire(siv0[b], tiv0[b], b)

    def group(g, carry):
        siv, tiv = idx_vecs(g)
        snx, tnx = idx_vecs(jnp.minimum(g + 1, _BPW // _L - 1))
        acc = jnp.zeros((_L,), jnp.float32)
        for b in range(_L):
            slot = b % _NB
            drain(slot)
            sv1 = sbufs[slot][pl.ds(0, _L)]
            sv2 = sbufs[slot][pl.ds(_L, _L)]
            tv1 = tbufs[slot][pl.ds(0, _L)]
            tv2 = tbufs[slot][pl.ds(_L, _L)]
            dot = jnp.sum(sv1 * tv1 + sv2 * tv2)
            acc = jnp.where(lane == b, dot, acc)

            if b < _NB:
                fire(siv[b + _NB], tiv[b + _NB], slot)
            else:
                rs, rt = snx[b - _NB], tnx[b - _NB]

                @pl.when(g < _BPW // _L - 1)
                def _():
                    fire(rs, rt, slot)

        outv[pl.ds(pl.multiple_of(g * _L, _L), _L)] = acc
        return carry

    lax.fori_loop(0, _BPW // _L, group, 0)
    pltpu.sync_copy(outv, out_hbm.at[pl.ds(pl.multiple_of(base, _L), _BPW)])


@functools.partial(
    pl.kernel,
    out_type=jax.ShapeDtypeStruct((_B,), jnp.float32),
    mesh=plsc.VectorSubcoreMesh(core_axis_name="c", subcore_axis_name="s"),
    scratch_types=[
        pltpu.VMEM((_BPW,), jnp.int32),
        pltpu.VMEM((_BPW,), jnp.int32),
        [pltpu.VMEM((_DIM,), jnp.float32)] * _NB,
        [pltpu.VMEM((_DIM,), jnp.float32)] * _NB,
        pltpu.VMEM((_BPW,), jnp.float32),
        pltpu.SemaphoreType.DMA((_NB,)),
        pltpu.SemaphoreType.DMA((_NB,)),
    ],
    compiler_params=pltpu.CompilerParams(needs_layout_passes=False),
)
def _dots(s_hbm, t_hbm, sinv_hbm, tinv_hbm, out_hbm, *scratch):
    _dots_body(s_hbm, t_hbm, sinv_hbm, tinv_hbm, out_hbm, *scratch)


def _loss_body(ip_ref, lab_ref, o_ref):
    x = lab_ref[...] * ip_ref[...]
    o_ref[0, 0] = -jnp.sum(jax.nn.log_sigmoid(x)) * (1.0 / _B)


_loss = pl.pallas_call(
    _loss_body,
    out_shape=jax.ShapeDtypeStruct((1, 1), jnp.float32),
    out_specs=pl.BlockSpec(memory_space=pltpu.MemorySpace.SMEM),
)


def kernel(source_node, target_node, label, nodes_embed, context_nodes_embed):
    iota = jnp.arange(_B, dtype=jnp.int32)
    ss, sp = lax.sort_key_val(source_node, iota)
    ts, tp = lax.sort_key_val(target_node, iota)
    _, sinv = lax.sort_key_val(sp, iota)
    _, tinv = lax.sort_key_val(tp, iota)
    srows = _gather_rows(ss, nodes_embed.T)
    trows = _gather_rows(ts, context_nodes_embed.T)
    ip = _dots(srows, trows, sinv, tinv)
    loss = _loss(ip.reshape(128, 128), label.reshape(128, 128))
    return loss.reshape(())


# speedup vs baseline: 9.5347x; 1.1662x over previous
import functools

import jax
import jax.numpy as jnp
from jax import lax
from jax.experimental import pallas as pl
from jax.experimental.pallas import tpu as pltpu
from jax.experimental.pallas import tpu_sc as plsc

_B = 16384
_DIM = 32
_NC = 2
_NS = 16
_NW = _NC * _NS
_BPW = _B // _NW
_NB = 8
_L = 16


def _sc_body(src_hbm, tgt_hbm, ns_hbm, ctx_hbm, out_hbm,
             sidx, tidx, sbufs, tbufs, outv, sems, semt):
    wid = lax.axis_index("s") * _NC + lax.axis_index("c")
    base = wid * _BPW

    pltpu.sync_copy(src_hbm.at[pl.ds(base, _BPW)], sidx)
    pltpu.sync_copy(tgt_hbm.at[pl.ds(base, _BPW)], tidx)

    lane = lax.iota(jnp.int32, _L)

    def fire(rs, rt, b):
        cs = pl.multiple_of(lax.shift_right_logical(rs, 7) * 128, 128)
        ct = pl.multiple_of(lax.shift_right_logical(rt, 7) * 128, 128)
        pltpu.async_copy(ns_hbm.at[:, pl.ds(cs, 128)], sbufs[b], sems.at[b])
        pltpu.async_copy(ctx_hbm.at[:, pl.ds(ct, 128)], tbufs[b], semt.at[b])

    def drain(b):
        pltpu.make_async_copy(ns_hbm.at[:, pl.ds(0, 128)], sbufs[b], sems.at[b]).wait()
        pltpu.make_async_copy(ctx_hbm.at[:, pl.ds(0, 128)], tbufs[b], semt.at[b]).wait()

    def idx_vecs(g):
        off = pl.multiple_of(g * _L, _L)
        return sidx[pl.ds(off, _L)], tidx[pl.ds(off, _L)]

    siv0, tiv0 = idx_vecs(0)
    for b in range(_NB):
        fire(siv0[b], tiv0[b], b)

    def group(g, carry):
        siv, tiv = idx_vecs(g)
        snx, tnx = idx_vecs(jnp.minimum(g + 1, _BPW // _L - 1))
        acc = jnp.zeros((_L,), jnp.float32)
        for b in range(_L):
            slot = b % _NB
            drain(slot)
            sl = jnp.full((_L,), siv[b] & 127, jnp.int32)
            tl = jnp.full((_L,), tiv[b] & 127, jnp.int32)
            sv1 = plsc.load_gather(sbufs[slot], [lane, sl])
            sv2 = plsc.load_gather(sbufs[slot], [lane + _L, sl])
            tv1 = plsc.load_gather(tbufs[slot], [lane, tl])
            tv2 = plsc.load_gather(tbufs[slot], [lane + _L, tl])
            dot = jnp.sum(sv1 * tv1 + sv2 * tv2)
            acc = jnp.where(lane == b, dot, acc)

            if b < _NB:
                fire(siv[b + _NB], tiv[b + _NB], slot)
            else:
                rs, rt = snx[b - _NB], tnx[b - _NB]

                @pl.when(g < _BPW // _L - 1)
                def _():
                    fire(rs, rt, slot)

        outv[pl.ds(pl.multiple_of(g * _L, _L), _L)] = acc
        return carry

    lax.fori_loop(0, _BPW // _L, group, 0)

    pltpu.sync_copy(outv, out_hbm.at[pl.ds(base, _BPW)])


@functools.partial(
    pl.kernel,
    out_type=jax.ShapeDtypeStruct((_B,), jnp.float32),
    mesh=plsc.VectorSubcoreMesh(core_axis_name="c", subcore_axis_name="s"),
    scratch_types=[
        pltpu.VMEM((_BPW,), jnp.int32),
        pltpu.VMEM((_BPW,), jnp.int32),
        [pltpu.VMEM((_DIM, 128), jnp.float32)] * _NB,
        [pltpu.VMEM((_DIM, 128), jnp.float32)] * _NB,
        pltpu.VMEM((_BPW,), jnp.float32),
        pltpu.SemaphoreType.DMA((_NB,)),
        pltpu.SemaphoreType.DMA((_NB,)),
    ],
    compiler_params=pltpu.CompilerParams(needs_layout_passes=False),
)
def _sc_dot(src_hbm, tgt_hbm, ns_hbm, ctx_hbm, out_hbm,
            sidx, tidx, sbufs, tbufs, outv, sems, semt):
    _sc_body(src_hbm, tgt_hbm, ns_hbm, ctx_hbm, out_hbm,
             sidx, tidx, sbufs, tbufs, outv, sems, semt)


def _loss_body(ip_ref, lab_ref, o_ref):
    x = lab_ref[...] * ip_ref[...]
    o_ref[0, 0] = -jnp.sum(jax.nn.log_sigmoid(x)) * (1.0 / _B)


_loss = pl.pallas_call(
    _loss_body,
    out_shape=jax.ShapeDtypeStruct((1, 1), jnp.float32),
    out_specs=pl.BlockSpec(memory_space=pltpu.MemorySpace.SMEM),
)


def kernel(source_node, target_node, label, nodes_embed, context_nodes_embed):
    ip = _sc_dot(source_node, target_node,
                 nodes_embed.T, context_nodes_embed.T)
    loss = _loss(ip.reshape(128, 128), label.reshape(128, 128))
    return loss.reshape(())
